# baseline (speedup 1.0000x reference)
import functools

import numpy as np
import jax
import jax.numpy as jnp
from jax import lax
from jax.experimental import pallas as pl
from jax.experimental.pallas import tpu as pltpu
from jax.experimental.pallas import tpu_sc as plsc

_B, _S = 4096, 200
_VOCAB = 100000
_ROT_A = (13, 15, 26, 6)
_ROT_B = (17, 29, 16, 24)
_TINY = np.float32(1.1754943508222875e-38)
_COLS = 3200
_BLOCK_ROWS = 64


def _np_threefry2x32(k0, k1, c0, c1):
    k0 = np.uint32(k0)
    k1 = np.uint32(k1)
    x0 = (np.asarray(c0, np.uint32) + k0).astype(np.uint32)
    x1 = (np.asarray(c1, np.uint32) + k1).astype(np.uint32)
    ks = (k0, k1, np.uint32(k0 ^ k1 ^ np.uint32(0x1BD11BDA)))
    for d in range(5):
        for r in (_ROT_A if d % 2 == 0 else _ROT_B):
            x0 = (x0 + x1).astype(np.uint32)
            x1 = (((x1 << np.uint32(r)) | (x1 >> np.uint32(32 - r))) ^ x0).astype(np.uint32)
        x0 = (x0 + ks[(d + 1) % 3]).astype(np.uint32)
        x1 = (x1 + ks[(d + 2) % 3] + np.uint32(d + 1)).astype(np.uint32)
    return x0, x1


def _derive_keys(seed):
    o0, o1 = _np_threefry2x32(0, seed, np.uint32([0, 0]), np.uint32([0, 1]))
    k_mask = (int(o0[0]), int(o1[0]))
    k2 = (int(o0[1]), int(o1[1]))
    p0, p1 = _np_threefry2x32(k2[0], k2[1], np.uint32([0, 0]), np.uint32([0, 1]))
    k_samp = (int(p0[1]), int(p1[1]))
    return k_mask, k_samp


_K_MASK, _K_SAMP = _derive_keys(42)


def _tf_bits(key, ctr):
    k0 = np.uint32(key[0])
    k1 = np.uint32(key[1])
    ks2 = np.uint32(k0 ^ k1 ^ np.uint32(0x1BD11BDA))
    ks = (k0, k1, ks2)
    x0 = jnp.full_like(ctr, k0)
    x1 = ctr + k1
    for d in range(5):
        for r in (_ROT_A if d % 2 == 0 else _ROT_B):
            x0 = x0 + x1
            x1 = ((x1 << np.uint32(r)) | (x1 >> np.uint32(32 - r))) ^ x0
        x0 = x0 + ks[(d + 1) % 3]
        x1 = x1 + ks[(d + 2) % 3] + np.uint32(d + 1)
    return x0 ^ x1


def _uniform(bits):
    return jax.lax.bitcast_convert_type(
        (bits >> np.uint32(9)) | np.uint32(0x3F800000), jnp.float32) - 1.0


def _body(off_ref, x_ref, o_ref):
    i = pl.program_id(0)
    row_off = off_ref[0, 0].astype(jnp.uint32)

    c1 = jax.lax.broadcasted_iota(jnp.uint32, (1, _COLS), 1)
    cdiv1 = (c1.astype(jnp.float32) * np.float32(1.0 / _S)).astype(jnp.int32
                                                                   ).astype(jnp.uint32)
    colterm = (c1 - cdiv1 * np.uint32(_S)) * np.uint32(2 * _B) + cdiv1 * np.uint32(2)
    r1 = jax.lax.broadcasted_iota(jnp.uint32, (_BLOCK_ROWS, 1), 0)
    gr1 = r1 + np.uint32(_BLOCK_ROWS) * i.astype(jnp.uint32)

    f0 = colterm + (gr1 * np.uint32(2 * (_COLS // _S)) + row_off * np.uint32(2))
    u0 = _uniform(_tf_bits(_K_MASK, f0))
    u1 = _uniform(_tf_bits(_K_MASK, f0 + np.uint32(1)))
    u2 = u0 * u0
    u4 = u2 * u2
    p9 = u4 * u4 * u0
    drop = u1 <= p9

    v = _tf_bits(_K_SAMP,
                 c1 + (gr1 * np.uint32(_COLS) + row_off * np.uint32(_S)))
    vi = jax.lax.bitcast_convert_type(v, jnp.int32)
    q = (v.astype(jnp.float32) * np.float32(1.0 / _VOCAB)).astype(jnp.int32)
    rr = vi - q * np.int32(_VOCAB)
    rr = jnp.where(rr < 0, rr + np.int32(_VOCAB), rr)
    rr = jnp.where(rr >= _VOCAB, rr - np.int32(_VOCAB), rr)

    o_ref[...] = jnp.where(drop, rr.astype(jnp.float32), x_ref[...])


def _shard_rows(x, row_off):
    rows, cols = x.shape
    dense_rows = rows * cols // _COLS
    xd = x.reshape(dense_rows, _COLS)
    off_arr = jnp.reshape(row_off.astype(jnp.int32), (1, 1))
    out = pl.pallas_call(
        _body,
        grid=(dense_rows // _BLOCK_ROWS,),
        in_specs=[
            pl.BlockSpec(memory_space=pltpu.SMEM),
            pl.BlockSpec((_BLOCK_ROWS, _COLS), lambda i: (i, 0)),
        ],
        out_specs=pl.BlockSpec((_BLOCK_ROWS, _COLS), lambda i: (i, 0)),
        out_shape=jax.ShapeDtypeStruct((dense_rows, _COLS), jnp.float32),
    )(off_arr, xd)
    return out.reshape(rows, cols)


_SC_ROWS = 512
_NC, _NS, _LANES = 2, 16, 16
_NW = _NC * _NS


def _sc_tail(x_flat, row0):
    elems = _SC_ROWS * _S
    per_w = elems // _NW
    mesh = plsc.VectorSubcoreMesh(core_axis_name="c", subcore_axis_name="s")

    @functools.partial(
        pl.kernel, mesh=mesh,
        out_type=jax.ShapeDtypeStruct((elems,), jnp.float32),
        scratch_types=[
            pltpu.VMEM((per_w,), jnp.float32),
            pltpu.VMEM((per_w,), jnp.float32),
        ],
    )
    def k(x_hbm, out_hbm, xin_v, xout_v):
        wid = lax.axis_index("s") * _NC + lax.axis_index("c")
        base = wid * per_w
        pltpu.sync_copy(x_hbm.at[pl.ds(base, per_w)], xin_v)
        gbase = np.int32(row0 * _S) + base

        def body(j, carry):
            off = j * np.int32(_LANES)
            gidx = gbase + off + lax.iota(jnp.int32, _LANES)
            gu = gidx.astype(jnp.uint32)
            bq = (gidx.astype(jnp.float32) * np.float32(1.0 / _S)
                  ).astype(jnp.int32).astype(jnp.uint32)
            s = gu - bq * np.uint32(_S)
            f0 = s * np.uint32(2 * _B) + bq * np.uint32(2)
            u0 = _uniform(_tf_bits(_K_MASK, f0))
            u1 = _uniform(_tf_bits(_K_MASK, f0 + np.uint32(1)))
            u2 = u0 * u0
            u4 = u2 * u2
            drop = u1 <= u4 * u4 * u0
            v = _tf_bits(_K_SAMP, gu)
            vi = jax.lax.bitcast_convert_type(v, jnp.int32)
            q = (v.astype(jnp.float32) * np.float32(1.0 / _VOCAB)).astype(jnp.int32)
            rr = vi - q * np.int32(_VOCAB)
            rr = jnp.where(rr < 0, rr + np.int32(_VOCAB), rr)
            rr = jnp.where(rr >= _VOCAB, rr - np.int32(_VOCAB), rr)
            xv = xin_v[pl.ds(off, _LANES)]
            xout_v[pl.ds(off, _LANES)] = jnp.where(drop, rr.astype(jnp.float32), xv)
            return carry

        lax.fori_loop(0, per_w // _LANES, body, jnp.int32(0))
        pltpu.sync_copy(xout_v, out_hbm.at[pl.ds(base, per_w)])

    return k(x_flat)


def kernel(inputs):
    return _shard_rows(inputs, jnp.int32(0))


# speedup vs baseline: 1.0005x; 1.0005x over previous
import numpy as np
import jax
import jax.numpy as jnp
from jax.experimental import pallas as pl
from jax.experimental.pallas import tpu as pltpu

_B, _S = 4096, 200
_VOCAB = 100000
_ROT_A = (13, 15, 26, 6)
_ROT_B = (17, 29, 16, 24)
_COLS = 3200
_BLOCK_ROWS = 64


def _np_threefry2x32(k0, k1, c0, c1):
    k0 = np.uint32(k0)
    k1 = np.uint32(k1)
    x0 = (np.asarray(c0, np.uint32) + k0).astype(np.uint32)
    x1 = (np.asarray(c1, np.uint32) + k1).astype(np.uint32)
    ks = (k0, k1, np.uint32(k0 ^ k1 ^ np.uint32(0x1BD11BDA)))
    for d in range(5):
        for r in (_ROT_A if d % 2 == 0 else _ROT_B):
            x0 = (x0 + x1).astype(np.uint32)
            x1 = (((x1 << np.uint32(r)) | (x1 >> np.uint32(32 - r))) ^ x0).astype(np.uint32)
        x0 = (x0 + ks[(d + 1) % 3]).astype(np.uint32)
        x1 = (x1 + ks[(d + 2) % 3] + np.uint32(d + 1)).astype(np.uint32)
    return x0, x1


def _derive_keys(seed):
    o0, o1 = _np_threefry2x32(0, seed, np.uint32([0, 0]), np.uint32([0, 1]))
    k_mask = (int(o0[0]), int(o1[0]))
    k2 = (int(o0[1]), int(o1[1]))
    p0, p1 = _np_threefry2x32(k2[0], k2[1], np.uint32([0, 0]), np.uint32([0, 1]))
    k_samp = (int(p0[1]), int(p1[1]))
    return k_mask, k_samp


_K_MASK, _K_SAMP = _derive_keys(42)


def _tf_bits(key, ctr):
    k0 = np.uint32(key[0])
    k1 = np.uint32(key[1])
    ks2 = np.uint32(k0 ^ k1 ^ np.uint32(0x1BD11BDA))
    ks = (k0, k1, ks2)
    x0 = jnp.full_like(ctr, k0)
    x1 = ctr + k1
    for d in range(5):
        for r in (_ROT_A if d % 2 == 0 else _ROT_B):
            x0 = x0 + x1
            x1 = ((x1 << np.uint32(r)) | (x1 >> np.uint32(32 - r))) ^ x0
        x0 = x0 + ks[(d + 1) % 3]
        x1 = x1 + ks[(d + 2) % 3] + np.uint32(d + 1)
    return x0 ^ x1


def _uniform(bits):
    return jax.lax.bitcast_convert_type(
        (bits >> np.uint32(9)) | np.uint32(0x3F800000), jnp.float32) - 1.0


def _body(off_ref, x_ref, o_ref):
    i = pl.program_id(0)
    row_off = off_ref[0, 0].astype(jnp.uint32)

    c1 = jax.lax.broadcasted_iota(jnp.uint32, (1, _COLS), 1)
    cdiv1 = (c1.astype(jnp.float32) * np.float32(1.0 / _S)).astype(jnp.int32
                                                                   ).astype(jnp.uint32)
    colterm = (c1 - cdiv1 * np.uint32(_S)) * np.uint32(2 * _B) + cdiv1 * np.uint32(2)
    r1 = jax.lax.broadcasted_iota(jnp.uint32, (_BLOCK_ROWS, 1), 0)
    gr1 = r1 + np.uint32(_BLOCK_ROWS) * i.astype(jnp.uint32)

    f0 = colterm + (gr1 * np.uint32(2 * (_COLS // _S)) + row_off * np.uint32(2))
    u0 = _uniform(_tf_bits(_K_MASK, f0))
    u1 = _uniform(_tf_bits(_K_MASK, f0 + np.uint32(1)))
    u2 = u0 * u0
    u4 = u2 * u2
    p9 = u4 * u4 * u0
    drop = u1 <= p9

    v = _tf_bits(_K_SAMP,
                 c1 + (gr1 * np.uint32(_COLS) + row_off * np.uint32(_S)))
    vi = jax.lax.bitcast_convert_type(v, jnp.int32)
    q = (v.astype(jnp.float32) * np.float32(1.0 / _VOCAB)).astype(jnp.int32)
    rr = vi - q * np.int32(_VOCAB)
    rr = jnp.where(rr < 0, rr + np.int32(_VOCAB), rr)
    rr = jnp.where(rr >= _VOCAB, rr - np.int32(_VOCAB), rr)

    o_ref[...] = jnp.where(drop, rr.astype(jnp.float32), x_ref[...])


def _shard_rows(x, row_off):
    rows, cols = x.shape
    dense_rows = rows * cols // _COLS
    xd = x.reshape(dense_rows, _COLS)
    off_arr = jnp.reshape(row_off.astype(jnp.int32), (1, 1))
    out = pl.pallas_call(
        _body,
        grid=(dense_rows // _BLOCK_ROWS,),
        in_specs=[
            pl.BlockSpec(memory_space=pltpu.SMEM),
            pl.BlockSpec((_BLOCK_ROWS, _COLS), lambda i: (i, 0)),
        ],
        out_specs=pl.BlockSpec((_BLOCK_ROWS, _COLS), lambda i: (i, 0)),
        out_shape=jax.ShapeDtypeStruct((dense_rows, _COLS), jnp.float32),
    )(off_arr, xd)
    return out.reshape(rows, cols)


def kernel(inputs):
    return _shard_rows(inputs, jnp.int32(0))
